# Initial kernel scaffold; baseline (speedup 1.0000x reference)
import jax
import jax.numpy as jnp
from jax.experimental import pallas as pl


def kernel(x, gamma, beta, w_fc0):
    raise NotImplementedError("write your pallas kernel here")


# speedup vs baseline: 2.0338x; 2.0338x over previous
import jax
import jax.numpy as jnp
from jax.experimental import pallas as pl
from jax.experimental.pallas import tpu as pltpu

_EPS = 1e-05


def _fused_block(x_ref, g_ref, b_ref, w_ref, o_ref, *, tc, d, gn_groups):
    cgn = d // gn_groups
    xb = x_ref[0]
    xg = xb.reshape(gn_groups, cgn, tc)
    s = jnp.sum(xg, axis=1, keepdims=True)
    ss = jnp.sum(xg * xg, axis=1, keepdims=True)
    mean = s * (1.0 / cgn)
    var = (ss - cgn * mean * mean) * (1.0 / (cgn - 1))
    inv = jax.lax.rsqrt(var + _EPS)
    net = ((xg - mean) * inv).reshape(d, tc)
    gamma = pltpu.repeat(g_ref[...], tc // 128, axis=1)
    beta = pltpu.repeat(b_ref[...], tc // 128, axis=1)
    h = jnp.maximum(net * gamma + beta, 0.0)
    o_ref[0] = xb + jnp.dot(w_ref[...], h, preferred_element_type=jnp.float32)


def kernel(x, gamma, beta, w_fc0):
    b, d, t = x.shape
    groups = 8
    gn_groups = groups * 4
    cg = d // groups

    wg = w_fc0.reshape(groups, cg, cg)
    w_bd = (wg[:, :, None, :] * jnp.eye(groups, dtype=w_fc0.dtype)[:, None, :, None])
    w_bd = w_bd.reshape(d, d)

    g2 = jnp.broadcast_to(gamma.reshape(d, 1), (d, 128))
    b2 = jnp.broadcast_to(beta.reshape(d, 1), (d, 128))

    tc = min(512, t)
    grid = (b, t // tc)

    import functools
    body = functools.partial(_fused_block, tc=tc, d=d, gn_groups=gn_groups)

    return pl.pallas_call(
        body,
        grid=grid,
        in_specs=[
            pl.BlockSpec((1, d, tc), lambda i, j: (i, 0, j)),
            pl.BlockSpec((d, 128), lambda i, j: (0, 0)),
            pl.BlockSpec((d, 128), lambda i, j: (0, 0)),
            pl.BlockSpec((d, d), lambda i, j: (0, 0)),
        ],
        out_specs=pl.BlockSpec((1, d, tc), lambda i, j: (i, 0, j)),
        out_shape=jax.ShapeDtypeStruct((b, d, t), x.dtype),
        compiler_params=pltpu.CompilerParams(
            dimension_semantics=("parallel", "parallel"),
        ),
    )(x, g2, b2, w_bd)
